# Initial kernel scaffold; baseline (speedup 1.0000x reference)
import jax
import jax.numpy as jnp
from jax.experimental import pallas as pl


def kernel(atom_features, pair_features, pair_split, atom_to_pair, W_AA, b_AA, W_PA, b_PA, W_A, b_A, W_AP, b_AP, W_PP, b_PP, W_P, b_P):
    raise NotImplementedError("write your pallas kernel here")


# speedup vs baseline: 1.5600x; 1.5600x over previous
import functools

import jax
import jax.numpy as jnp
from jax import lax
from jax.experimental import pallas as pl
from jax.experimental.pallas import tpu as pltpu
from jax.experimental.pallas import tpu_sc as plsc

NA = 50000
NP = 800000
FA = 75
FP = 14
H = 50
HP = 64
NC = 2
NS = 16
NW = NC * NS

f32 = jnp.float32


def _atom_pre_body(feat, w, b, aa_ref, x_ref):
    y = jnp.dot(feat[...], w[...], preferred_element_type=f32) + b[...]
    aa_ref[...] = jnp.maximum(y[:, :HP], 0.0)
    x_ref[...] = y[:, HP:]


def _atom_pre(feat, wcat, bcat):
    RA = 2000
    return pl.pallas_call(
        _atom_pre_body,
        grid=(NA // RA,),
        in_specs=[
            pl.BlockSpec((RA, FA), lambda i: (i, 0)),
            pl.BlockSpec((FA, 3 * HP), lambda i: (0, 0)),
            pl.BlockSpec((1, 3 * HP), lambda i: (0, 0)),
        ],
        out_specs=[
            pl.BlockSpec((RA, HP), lambda i: (i, 0)),
            pl.BlockSpec((RA, 2 * HP), lambda i: (i, 0)),
        ],
        out_shape=[
            jax.ShapeDtypeStruct((NA, HP), f32),
            jax.ShapeDtypeStruct((NA, 2 * HP), f32),
        ],
    )(feat, wcat, bcat)


_BP = 2000
_NBP = NP // _BP


def _pair_pre_body(pf, w, b, wp2, c_ref, pp_ref, carry):
    i = pl.program_id(0)

    @pl.when(i == 0)
    def _():
        carry[...] = jnp.zeros((8, HP), f32)

    t = jnp.maximum(jnp.dot(pf[...], w[...], preferred_element_type=f32) + b[...], 0.0)
    pa = t[:, :HP]
    pp_ref[...] = jnp.dot(t[:, HP:], wp2[...], preferred_element_type=f32)
    cr = carry[0:1, :]
    c_ref[:, HP:] = jnp.zeros((_BP, HP), f32)

    @pl.when(i < _NBP)
    def _():
        rows = lax.broadcasted_iota(jnp.int32, (_BP, HP), 0)
        inc = pa
        d = 1
        while d < _BP:
            inc = inc + jnp.where(rows >= d, pltpu.roll(inc, d, 0), 0.0)
            d *= 2
        c_ref[:, :HP] = inc - pa + cr
        carry[0:1, :] = cr + inc[_BP - 1:_BP, :]

    @pl.when(i == _NBP)
    def _():
        c_ref[:, :HP] = jnp.broadcast_to(cr, (_BP, HP))


def _pair_pre(pf, w2cat, b2cat, wp2):
    return pl.pallas_call(
        _pair_pre_body,
        grid=(_NBP + 1,),
        in_specs=[
            pl.BlockSpec((_BP, FP), lambda i: (jnp.minimum(i, _NBP - 1), 0)),
            pl.BlockSpec((FP, 2 * HP), lambda i: (0, 0)),
            pl.BlockSpec((1, 2 * HP), lambda i: (0, 0)),
            pl.BlockSpec((HP, HP), lambda i: (0, 0)),
        ],
        out_specs=[
            pl.BlockSpec((_BP, 2 * HP), lambda i: (i, 0)),
            pl.BlockSpec((_BP, HP), lambda i: (jnp.minimum(i, _NBP - 1), 0)),
        ],
        out_shape=[
            jax.ShapeDtypeStruct((NP + _BP, 2 * HP), f32),
            jax.ShapeDtypeStruct((NP, HP), f32),
        ],
        scratch_shapes=[pltpu.VMEM((8, HP), f32)],
    )(pf, w2cat, b2cat, wp2)


def _pair_out_body(ap, pp, wp1, bp, out_ref):
    y = jnp.dot(ap[...], wp1[...], preferred_element_type=f32)
    out_ref[...] = jnp.maximum(y + pp[:, :H] + bp[...], 0.0)


def _pair_out(ap, pp, wp1, bp):
    BP = 2000
    return pl.pallas_call(
        _pair_out_body,
        grid=(NP // BP,),
        in_specs=[
            pl.BlockSpec((BP, HP), lambda i: (i, 0)),
            pl.BlockSpec((BP, HP), lambda i: (i, 0)),
            pl.BlockSpec((HP, H), lambda i: (0, 0)),
            pl.BlockSpec((1, H), lambda i: (0, 0)),
        ],
        out_specs=pl.BlockSpec((BP, H), lambda i: (i, 0)),
        out_shape=jax.ShapeDtypeStruct((NP, H), f32),
    )(ap, pp, wp1, bp)


def _atom_out_body(aa, seg, wa1, wa2, ba, out_ref):
    y = jnp.dot(aa[...], wa1[...], preferred_element_type=f32)
    y += jnp.dot(seg[...], wa2[...], preferred_element_type=f32)
    out_ref[...] = jnp.maximum(y + ba[...], 0.0)


def _atom_out(aa, seg, wa1, wa2, ba):
    RA = 2000
    return pl.pallas_call(
        _atom_out_body,
        grid=(NA // RA,),
        in_specs=[
            pl.BlockSpec((RA, HP), lambda i: (i, 0)),
            pl.BlockSpec((RA, HP), lambda i: (i, 0)),
            pl.BlockSpec((HP, H), lambda i: (0, 0)),
            pl.BlockSpec((HP, H), lambda i: (0, 0)),
            pl.BlockSpec((1, H), lambda i: (0, 0)),
        ],
        out_specs=pl.BlockSpec((RA, H), lambda i: (i, 0)),
        out_shape=jax.ShapeDtypeStruct((NA, H), f32),
    )(aa, seg, wa1, wa2, ba)


_MESH = plsc.VectorSubcoreMesh(core_axis_name="c", subcore_axis_name="s")

_PPW = NP // NW
_CH = 128
_NCHK = -(-_PPW // _CH)
_GPT = 1568
_NAp = NW * _GPT
_GNC = -(-_GPT // _CH)


@functools.partial(
    pl.kernel,
    mesh=_MESH,
    out_type=(
        jax.ShapeDtypeStruct((NP, HP), f32),
        jax.ShapeDtypeStruct((_NAp, HP), f32),
    ),
    scratch_types=[
        pltpu.VMEM((_CH,), jnp.int32),
        pltpu.VMEM((_CH,), jnp.int32),
        pltpu.VMEM((_CH, 2 * HP), f32),
        pltpu.VMEM((_CH, 2 * HP), f32),
        pltpu.VMEM((_CH, HP), f32),
        pltpu.SemaphoreType.DMA,
    ],
)
def _sc_pairs_kernel(x_hbm, ai_hbm, aj_hbm, cext_hbm, e0_hbm, e1_hbm,
                     ap_hbm, seg_hbm, ii, jj, xi, xj, ov, sem):
    c = lax.axis_index("c")
    s = lax.axis_index("s")
    wid = s * NC + c
    base = wid * _PPW

    def compute():
        def body(k, carry):
            for q in range(HP // 16):
                sl = pl.ds(q * 16, 16)
                sl2 = pl.ds(HP + q * 16, 16)
                s1 = xi[k, sl] + xj[k, sl2]
                s2 = xj[k, sl] + xi[k, sl2]
                ov[k, sl] = jnp.maximum(s1, 0.0) + jnp.maximum(s2, 0.0)
            return carry
        lax.fori_loop(0, _CH, body, 0, unroll=2)

    def chunk(it, carry):
        off = base + jnp.minimum(it * _CH, _PPW - _CH)
        pltpu.sync_copy(ai_hbm.at[pl.ds(off, _CH)], ii)
        pltpu.sync_copy(aj_hbm.at[pl.ds(off, _CH)], jj)
        ca = pltpu.async_copy(x_hbm.at[ii], xi, sem)
        cb = pltpu.async_copy(x_hbm.at[jj], xj, sem)
        ca.wait()
        cb.wait()
        compute()
        pltpu.sync_copy(ov, ap_hbm.at[pl.ds(off, _CH)])
        return carry

    lax.fori_loop(0, _NCHK, chunk, 0)

    gbase = wid * _GPT

    def gchunk(it, carry):
        off = gbase + jnp.minimum(it * _CH, _GPT - _CH)
        pltpu.sync_copy(e0_hbm.at[pl.ds(off, _CH)], ii)
        pltpu.sync_copy(e1_hbm.at[pl.ds(off, _CH)], jj)
        ca = pltpu.async_copy(cext_hbm.at[ii], xi, sem)
        cb = pltpu.async_copy(cext_hbm.at[jj], xj, sem)
        ca.wait()
        cb.wait()

        def body(k, carry2):
            for q in range(HP // 16):
                sl = pl.ds(q * 16, 16)
                ov[k, sl] = xj[k, sl] - xi[k, sl]
            return carry2
        lax.fori_loop(0, _CH, body, 0, unroll=2)
        pltpu.sync_copy(ov, seg_hbm.at[pl.ds(off, _CH)])
        return carry

    lax.fori_loop(0, _GNC, gchunk, 0)


def kernel(atom_features, pair_features, pair_split, atom_to_pair,
           W_AA, b_AA, W_PA, b_PA, W_A, b_A,
           W_AP, b_AP, W_PP, b_PP, W_P, b_P):
    wcat = jnp.zeros((FA, 3 * HP), f32)
    wcat = wcat.at[:, 0:H].set(W_AA)
    wcat = wcat.at[:, HP:HP + H].set(W_AP[:FA])
    wcat = wcat.at[:, 2 * HP:2 * HP + H].set(W_AP[FA:])
    bcat = jnp.zeros((1, 3 * HP), f32)
    bcat = bcat.at[0, 0:H].set(b_AA)
    bcat = bcat.at[0, HP:HP + H].set(0.5 * b_AP)
    bcat = bcat.at[0, 2 * HP:2 * HP + H].set(0.5 * b_AP)

    w2cat = jnp.zeros((FP, 2 * HP), f32)
    w2cat = w2cat.at[:, 0:H].set(W_PA)
    w2cat = w2cat.at[:, HP:HP + H].set(W_PP)
    b2cat = jnp.zeros((1, 2 * HP), f32)
    b2cat = b2cat.at[0, 0:H].set(b_PA)
    b2cat = b2cat.at[0, HP:HP + H].set(b_PP)

    wp2 = jnp.zeros((HP, HP), f32)
    wp2 = wp2.at[:H, :H].set(W_P[H:])
    wp1 = jnp.zeros((HP, H), f32)
    wp1 = wp1.at[:H].set(W_P[:H])
    bp = b_P.reshape(1, H)

    wa1 = jnp.zeros((HP, H), f32)
    wa1 = wa1.at[:H].set(W_A[:H])
    wa2 = jnp.zeros((HP, H), f32)
    wa2 = wa2.at[:H].set(W_A[H:])
    ba = b_A.reshape(1, H)

    ai = atom_to_pair[:, 0].astype(jnp.int32)
    aj = atom_to_pair[:, 1].astype(jnp.int32)
    split = pair_split.astype(jnp.int32)

    e = jnp.searchsorted(split, jnp.arange(NA + 1, dtype=jnp.int32),
                         side="left").astype(jnp.int32)
    e0p = jnp.zeros((_NAp,), jnp.int32).at[:NA].set(e[:NA])
    e1p = jnp.zeros((_NAp,), jnp.int32).at[:NA].set(e[1:])

    aa, x = _atom_pre(atom_features, wcat, bcat)
    cext, pp = _pair_pre(pair_features, w2cat, b2cat, wp2)
    ap, seg = _sc_pairs_kernel(x, ai, aj, cext, e0p, e1p)
    P = _pair_out(ap, pp, wp1, bp)
    A = _atom_out(aa, seg, wa1, wa2, ba)
    return (A, P)


# speedup vs baseline: 3.7664x; 2.4144x over previous
import functools

import jax
import jax.numpy as jnp
from jax import lax
from jax.experimental import pallas as pl
from jax.experimental.pallas import tpu as pltpu
from jax.experimental.pallas import tpu_sc as plsc

NA = 50000
NP = 800000
FA = 75
FP = 14
H = 50
HP = 64
NC = 2
NS = 16
NW = NC * NS

f32 = jnp.float32

_GDN = lax.GatherDimensionNumbers(offset_dims=(), collapsed_slice_dims=(0,),
                                  start_index_map=(0,))


def _take16(x, idx):
    return lax.gather(x, idx[:, None], _GDN, (1,),
                      mode=lax.GatherScatterMode.PROMISE_IN_BOUNDS)


def _atom_pre_body(feat, w, b, aa_ref, x_ref):
    y = jnp.dot(feat[...], w[...], preferred_element_type=f32) + b[...]
    aa_ref[...] = jnp.maximum(y[:, :HP], 0.0)
    x_ref[...] = y[:, HP:]


def _atom_pre(feat, wcat, bcat):
    RA = 2000
    return pl.pallas_call(
        _atom_pre_body,
        grid=(NA // RA,),
        in_specs=[
            pl.BlockSpec((RA, FA), lambda i: (i, 0)),
            pl.BlockSpec((FA, 3 * HP), lambda i: (0, 0)),
            pl.BlockSpec((1, 3 * HP), lambda i: (0, 0)),
        ],
        out_specs=[
            pl.BlockSpec((RA, HP), lambda i: (i, 0)),
            pl.BlockSpec((RA, 2 * HP), lambda i: (i, 0)),
        ],
        out_shape=[
            jax.ShapeDtypeStruct((NA, HP), f32),
            jax.ShapeDtypeStruct((NA, 2 * HP), f32),
        ],
    )(feat, wcat, bcat)


_BP = 2000
_NBP = NP // _BP


def _pair_pre_body(pf, w, b, wp2, c_ref, pp_ref, carry):
    i = pl.program_id(0)

    @pl.when(i == 0)
    def _():
        carry[...] = jnp.zeros((8, HP), f32)

    t = jnp.maximum(jnp.dot(pf[...], w[...], preferred_element_type=f32) + b[...], 0.0)
    pa = t[:, :HP]
    pp_ref[...] = jnp.dot(t[:, HP:], wp2[...], preferred_element_type=f32)
    cr = carry[0:1, :]
    c_ref[:, HP:] = jnp.zeros((_BP, HP), f32)

    @pl.when(i < _NBP)
    def _():
        rows = lax.broadcasted_iota(jnp.int32, (_BP, HP), 0)
        inc = pa
        d = 1
        while d < _BP:
            inc = inc + jnp.where(rows >= d, pltpu.roll(inc, d, 0), 0.0)
            d *= 2
        c_ref[:, :HP] = inc - pa + cr
        carry[0:1, :] = cr + inc[_BP - 1:_BP, :]

    @pl.when(i == _NBP)
    def _():
        c_ref[:, :HP] = jnp.broadcast_to(cr, (_BP, HP))


def _pair_pre(pf, w2cat, b2cat, wp2):
    return pl.pallas_call(
        _pair_pre_body,
        grid=(_NBP + 1,),
        in_specs=[
            pl.BlockSpec((_BP, FP), lambda i: (jnp.minimum(i, _NBP - 1), 0)),
            pl.BlockSpec((FP, 2 * HP), lambda i: (0, 0)),
            pl.BlockSpec((1, 2 * HP), lambda i: (0, 0)),
            pl.BlockSpec((HP, HP), lambda i: (0, 0)),
        ],
        out_specs=[
            pl.BlockSpec((_BP, 2 * HP), lambda i: (i, 0)),
            pl.BlockSpec((_BP, HP), lambda i: (jnp.minimum(i, _NBP - 1), 0)),
        ],
        out_shape=[
            jax.ShapeDtypeStruct((NP + _BP, 2 * HP), f32),
            jax.ShapeDtypeStruct((NP, HP), f32),
        ],
        scratch_shapes=[pltpu.VMEM((8, HP), f32)],
    )(pf, w2cat, b2cat, wp2)


def _pair_out_body(ap, pp, wp1, bp, out_ref):
    y = jnp.dot(ap[...], wp1[...], preferred_element_type=f32)
    out_ref[...] = jnp.maximum(y + pp[:, :H] + bp[...], 0.0)


def _pair_out(ap, pp, wp1, bp):
    BP = 2000
    return pl.pallas_call(
        _pair_out_body,
        grid=(NP // BP,),
        in_specs=[
            pl.BlockSpec((BP, HP), lambda i: (i, 0)),
            pl.BlockSpec((BP, HP), lambda i: (i, 0)),
            pl.BlockSpec((HP, H), lambda i: (0, 0)),
            pl.BlockSpec((1, H), lambda i: (0, 0)),
        ],
        out_specs=pl.BlockSpec((BP, H), lambda i: (i, 0)),
        out_shape=jax.ShapeDtypeStruct((NP, H), f32),
    )(ap, pp, wp1, bp)


def _atom_out_body(aa, seg, wa1, wa2, ba, out_ref):
    y = jnp.dot(aa[...], wa1[...], preferred_element_type=f32)
    y += jnp.dot(seg[...], wa2[...], preferred_element_type=f32)
    out_ref[...] = jnp.maximum(y + ba[...], 0.0)


def _atom_out(aa, seg, wa1, wa2, ba):
    RA = 2000
    return pl.pallas_call(
        _atom_out_body,
        grid=(NA // RA,),
        in_specs=[
            pl.BlockSpec((RA, HP), lambda i: (i, 0)),
            pl.BlockSpec((RA, HP), lambda i: (i, 0)),
            pl.BlockSpec((HP, H), lambda i: (0, 0)),
            pl.BlockSpec((HP, H), lambda i: (0, 0)),
            pl.BlockSpec((1, H), lambda i: (0, 0)),
        ],
        out_specs=pl.BlockSpec((RA, H), lambda i: (i, 0)),
        out_shape=jax.ShapeDtypeStruct((NA, H), f32),
    )(aa, seg, wa1, wa2, ba)


_MESH = plsc.VectorSubcoreMesh(core_axis_name="c", subcore_axis_name="s")

_PPW = NP // NW
_CH = 128
_NCHK = -(-_PPW // _CH)
_GPT = 1568
_NAp = NW * _GPT
_GNC = -(-_GPT // _CH)
_T2N = NP // 16
_NQV = _GPT // 16 + 1
_SR = NP // 128


@functools.partial(
    pl.kernel,
    mesh=_MESH,
    compiler_params=pltpu.CompilerParams(needs_layout_passes=False),
    out_type=(
        jax.ShapeDtypeStruct((NP, HP), f32),
        jax.ShapeDtypeStruct((_NAp, HP), f32),
    ),
    scratch_types=[
        pltpu.VMEM((_CH,), jnp.int32),
        pltpu.VMEM((_CH,), jnp.int32),
        pltpu.VMEM((_CH, 2 * HP), f32),
        pltpu.VMEM((_CH, 2 * HP), f32),
        pltpu.VMEM((_CH, HP), f32),
        pltpu.VMEM((_T2N,), jnp.int32),
        pltpu.VMEM((_NQV * 16,), jnp.int32),
        pltpu.VMEM((16,), jnp.int32),
        pltpu.VMEM((16, 128), jnp.int32),
        pltpu.SemaphoreType.DMA,
    ],
)
def _sc_pairs_kernel(x_hbm, ai_hbm, aj_hbm, cext_hbm, t2_hbm, sr_hbm,
                     ap_hbm, seg_hbm, ii, jj, xi, xj, ov, t2v, ebuf,
                     rowi, rowb, sem):
    c = lax.axis_index("c")
    s = lax.axis_index("s")
    wid = s * NC + c
    base = wid * _PPW

    def compute():
        def body(k, carry):
            for q in range(HP // 16):
                sl = pl.ds(q * 16, 16)
                sl2 = pl.ds(HP + q * 16, 16)
                s1 = xi[k, sl] + xj[k, sl2]
                s2 = xj[k, sl] + xi[k, sl2]
                ov[k, sl] = jnp.maximum(s1, 0.0) + jnp.maximum(s2, 0.0)
            return carry
        lax.fori_loop(0, _CH, body, 0, unroll=2)

    def chunk(it, carry):
        off = base + jnp.minimum(it * _CH, _PPW - _CH)
        pltpu.sync_copy(ai_hbm.at[pl.ds(off, _CH)], ii)
        pltpu.sync_copy(aj_hbm.at[pl.ds(off, _CH)], jj)
        ca = pltpu.async_copy(x_hbm.at[ii], xi, sem)
        cb = pltpu.async_copy(x_hbm.at[jj], xj, sem)
        ca.wait()
        cb.wait()
        compute()
        pltpu.sync_copy(ov, ap_hbm.at[pl.ds(off, _CH)])
        return carry

    lax.fori_loop(0, _NCHK, chunk, 0)

    gbase = wid * _GPT
    pltpu.sync_copy(t2_hbm, t2v)
    lanes = lax.iota(jnp.int32, 16)

    def equery(v, carry):
        s_vec = gbase + v * 16 + lanes
        j = jnp.zeros((16,), jnp.int32)
        step = 1 << 15
        while step > 0:
            nj = j + step
            idx = jnp.minimum(nj - 1, _T2N - 1)
            val = plsc.load_gather(t2v, [idx])
            ok = (nj <= _T2N) & (val < s_vec)
            j = jnp.where(ok, nj, j)
            step >>= 1
        base16 = jnp.maximum(16 * j - 16, 0)
        rowi[...] = base16 >> 7
        pltpu.async_copy(sr_hbm.at[rowi], rowb, sem).wait()
        col0 = base16 & 127
        acc = jnp.zeros((16,), jnp.int32)
        for k in range(16):
            rk = jnp.full((16,), k, jnp.int32)
            colk = _take16(col0, rk)
            sk = _take16(s_vec, rk)
            bk = _take16(base16, rk)
            win = plsc.load_gather(rowb, [rk, colk + lanes])
            cnt = plsc.all_reduce_population_count(win < sk)
            acc = jnp.where(lanes == rk, bk + cnt, acc)
        ebuf[pl.ds(v * 16, 16)] = acc
        return carry

    lax.fori_loop(0, _NQV, equery, 0)

    def gchunk(it, carry):
        loff = jnp.minimum(it * _CH, _GPT - _CH)
        for q in range(_CH // 16):
            sl = pl.ds(q * 16, 16)
            ii[sl] = ebuf[pl.ds(loff + q * 16, 16)]
            jj[sl] = ebuf[pl.ds(loff + q * 16 + 1, 16)]
        ca = pltpu.async_copy(cext_hbm.at[ii], xi, sem)
        cb = pltpu.async_copy(cext_hbm.at[jj], xj, sem)
        ca.wait()
        cb.wait()

        def body(k, carry2):
            for q in range(HP // 16):
                sl = pl.ds(q * 16, 16)
                ov[k, sl] = xj[k, sl] - xi[k, sl]
            return carry2
        lax.fori_loop(0, _CH, body, 0, unroll=2)
        pltpu.sync_copy(ov, seg_hbm.at[pl.ds(gbase + loff, _CH)])
        return carry

    lax.fori_loop(0, _GNC, gchunk, 0)


def kernel(atom_features, pair_features, pair_split, atom_to_pair,
           W_AA, b_AA, W_PA, b_PA, W_A, b_A,
           W_AP, b_AP, W_PP, b_PP, W_P, b_P):
    wcat = jnp.zeros((FA, 3 * HP), f32)
    wcat = wcat.at[:, 0:H].set(W_AA)
    wcat = wcat.at[:, HP:HP + H].set(W_AP[:FA])
    wcat = wcat.at[:, 2 * HP:2 * HP + H].set(W_AP[FA:])
    bcat = jnp.zeros((1, 3 * HP), f32)
    bcat = bcat.at[0, 0:H].set(b_AA)
    bcat = bcat.at[0, HP:HP + H].set(0.5 * b_AP)
    bcat = bcat.at[0, 2 * HP:2 * HP + H].set(0.5 * b_AP)

    w2cat = jnp.zeros((FP, 2 * HP), f32)
    w2cat = w2cat.at[:, 0:H].set(W_PA)
    w2cat = w2cat.at[:, HP:HP + H].set(W_PP)
    b2cat = jnp.zeros((1, 2 * HP), f32)
    b2cat = b2cat.at[0, 0:H].set(b_PA)
    b2cat = b2cat.at[0, HP:HP + H].set(b_PP)

    wp2 = jnp.zeros((HP, HP), f32)
    wp2 = wp2.at[:H, :H].set(W_P[H:])
    wp1 = jnp.zeros((HP, H), f32)
    wp1 = wp1.at[:H].set(W_P[:H])
    bp = b_P.reshape(1, H)

    wa1 = jnp.zeros((HP, H), f32)
    wa1 = wa1.at[:H].set(W_A[:H])
    wa2 = jnp.zeros((HP, H), f32)
    wa2 = wa2.at[:H].set(W_A[H:])
    ba = b_A.reshape(1, H)

    ai = atom_to_pair[:, 0].astype(jnp.int32)
    aj = atom_to_pair[:, 1].astype(jnp.int32)
    split = pair_split.astype(jnp.int32)

    t2 = split[::16]
    sr = split.reshape(_SR, 128)

    aa, x = _atom_pre(atom_features, wcat, bcat)
    cext, pp = _pair_pre(pair_features, w2cat, b2cat, wp2)
    ap, seg = _sc_pairs_kernel(x, ai, aj, cext, t2, sr)
    P = _pair_out(ap, pp, wp1, bp)
    A = _atom_out(aa, seg, wa1, wa2, ba)
    return (A, P)


# speedup vs baseline: 4.4875x; 1.1915x over previous
import functools

import jax
import jax.numpy as jnp
from jax import lax
from jax.experimental import pallas as pl
from jax.experimental.pallas import tpu as pltpu
from jax.experimental.pallas import tpu_sc as plsc

NA = 50000
NP = 800000
FA = 75
FP = 14
H = 50
HP = 64
NC = 2
NS = 16
NW = NC * NS

f32 = jnp.float32

_GDN = lax.GatherDimensionNumbers(offset_dims=(), collapsed_slice_dims=(0,),
                                  start_index_map=(0,))


def _take16(x, idx):
    return lax.gather(x, idx[:, None], _GDN, (1,),
                      mode=lax.GatherScatterMode.PROMISE_IN_BOUNDS)


def _atom_pre_body(feat, w, b, aa_ref, x_ref):
    y = jnp.dot(feat[...], w[...], preferred_element_type=f32) + b[...]
    aa_ref[...] = jnp.maximum(y[:, :HP], 0.0)
    x_ref[...] = y[:, HP:]


def _atom_pre(feat, wcat, bcat):
    RA = 2000
    return pl.pallas_call(
        _atom_pre_body,
        grid=(NA // RA,),
        in_specs=[
            pl.BlockSpec((RA, FA), lambda i: (i, 0)),
            pl.BlockSpec((FA, 3 * HP), lambda i: (0, 0)),
            pl.BlockSpec((1, 3 * HP), lambda i: (0, 0)),
        ],
        out_specs=[
            pl.BlockSpec((RA, HP), lambda i: (i, 0)),
            pl.BlockSpec((RA, 2 * HP), lambda i: (i, 0)),
        ],
        out_shape=[
            jax.ShapeDtypeStruct((NA, HP), f32),
            jax.ShapeDtypeStruct((NA, 2 * HP), f32),
        ],
    )(feat, wcat, bcat)


_BP = 2000
_NBP = NP // _BP


def _pair_pre_body(pf, w, b, wp2, c_ref, pp_ref, carry):
    i = pl.program_id(0)

    @pl.when(i == 0)
    def _():
        carry[...] = jnp.zeros((8, HP), f32)

    t = jnp.maximum(jnp.dot(pf[...], w[...], preferred_element_type=f32) + b[...], 0.0)
    pa = t[:, :HP]
    pp_ref[...] = jnp.dot(t[:, HP:], wp2[...], preferred_element_type=f32)
    cr = carry[0:1, :]
    c_ref[:, HP:] = jnp.zeros((_BP, HP), f32)

    @pl.when(i < _NBP)
    def _():
        rows = lax.broadcasted_iota(jnp.int32, (_BP, HP), 0)
        inc = pa
        d = 1
        while d < _BP:
            inc = inc + jnp.where(rows >= d, pltpu.roll(inc, d, 0), 0.0)
            d *= 2
        c_ref[:, :HP] = inc - pa + cr
        carry[0:1, :] = cr + inc[_BP - 1:_BP, :]

    @pl.when(i == _NBP)
    def _():
        c_ref[:, :HP] = jnp.broadcast_to(cr, (_BP, HP))


def _pair_pre(pf, w2cat, b2cat, wp2):
    return pl.pallas_call(
        _pair_pre_body,
        grid=(_NBP + 1,),
        in_specs=[
            pl.BlockSpec((_BP, FP), lambda i: (jnp.minimum(i, _NBP - 1), 0)),
            pl.BlockSpec((FP, 2 * HP), lambda i: (0, 0)),
            pl.BlockSpec((1, 2 * HP), lambda i: (0, 0)),
            pl.BlockSpec((HP, HP), lambda i: (0, 0)),
        ],
        out_specs=[
            pl.BlockSpec((_BP, 2 * HP), lambda i: (i, 0)),
            pl.BlockSpec((_BP, HP), lambda i: (jnp.minimum(i, _NBP - 1), 0)),
        ],
        out_shape=[
            jax.ShapeDtypeStruct((NP + _BP, 2 * HP), f32),
            jax.ShapeDtypeStruct((NP, HP), f32),
        ],
        scratch_shapes=[pltpu.VMEM((8, HP), f32)],
    )(pf, w2cat, b2cat, wp2)


def _pair_out_body(ap, pp, wp1, bp, out_ref):
    y = jnp.dot(ap[...], wp1[...], preferred_element_type=f32)
    out_ref[...] = jnp.maximum(y + pp[:, :H] + bp[...], 0.0)


def _pair_out(ap, pp, wp1, bp):
    BP = 2000
    return pl.pallas_call(
        _pair_out_body,
        grid=(NP // BP,),
        in_specs=[
            pl.BlockSpec((BP, HP), lambda i: (i, 0)),
            pl.BlockSpec((BP, HP), lambda i: (i, 0)),
            pl.BlockSpec((HP, H), lambda i: (0, 0)),
            pl.BlockSpec((1, H), lambda i: (0, 0)),
        ],
        out_specs=pl.BlockSpec((BP, H), lambda i: (i, 0)),
        out_shape=jax.ShapeDtypeStruct((NP, H), f32),
    )(ap, pp, wp1, bp)


def _atom_out_body(aa, seg, wa1, wa2, ba, out_ref):
    y = jnp.dot(aa[...], wa1[...], preferred_element_type=f32)
    y += jnp.dot(seg[...], wa2[...], preferred_element_type=f32)
    out_ref[...] = jnp.maximum(y + ba[...], 0.0)


def _atom_out(aa, seg, wa1, wa2, ba):
    RA = 2000
    return pl.pallas_call(
        _atom_out_body,
        grid=(NA // RA,),
        in_specs=[
            pl.BlockSpec((RA, HP), lambda i: (i, 0)),
            pl.BlockSpec((RA, HP), lambda i: (i, 0)),
            pl.BlockSpec((HP, H), lambda i: (0, 0)),
            pl.BlockSpec((HP, H), lambda i: (0, 0)),
            pl.BlockSpec((1, H), lambda i: (0, 0)),
        ],
        out_specs=pl.BlockSpec((RA, H), lambda i: (i, 0)),
        out_shape=jax.ShapeDtypeStruct((NA, H), f32),
    )(aa, seg, wa1, wa2, ba)


_MESH = plsc.VectorSubcoreMesh(core_axis_name="c", subcore_axis_name="s")

_PPW = NP // NW
_CH = 128
_SC8 = 8
_SCP = _CH * _SC8
_NSC = -(-_PPW // _SCP)
_GPT = 1568
_NAp = NW * _GPT
_GNC = -(-_GPT // _CH)
_T2S = 32
_T2N = NP // _T2S
_NQV = _GPT // 16 + 1
_SR = NP // 128


@functools.partial(
    pl.kernel,
    mesh=_MESH,
    compiler_params=pltpu.CompilerParams(needs_layout_passes=False),
    out_type=(
        jax.ShapeDtypeStruct((NP, HP), f32),
        jax.ShapeDtypeStruct((_NAp, HP), f32),
    ),
    scratch_types=[
        pltpu.VMEM((_SCP,), jnp.int32),
        pltpu.VMEM((_SCP,), jnp.int32),
        pltpu.VMEM((_CH,), jnp.int32),
        pltpu.VMEM((_CH,), jnp.int32),
        pltpu.VMEM((_CH, 2 * HP), f32),
        pltpu.VMEM((_CH, 2 * HP), f32),
        pltpu.VMEM((_CH, 2 * HP), f32),
        pltpu.VMEM((_CH, 2 * HP), f32),
        pltpu.VMEM((_CH, HP), f32),
        pltpu.VMEM((_CH, HP), f32),
        pltpu.VMEM((_T2N,), jnp.int32),
        pltpu.VMEM((_NQV * 16,), jnp.int32),
        pltpu.VMEM((16,), jnp.int32),
        pltpu.VMEM((16, 128), jnp.int32),
        pltpu.SemaphoreType.DMA,
        pltpu.SemaphoreType.DMA,
        pltpu.SemaphoreType.DMA,
        pltpu.SemaphoreType.DMA,
    ],
)
def _sc_pairs_kernel(x_hbm, ai_hbm, aj_hbm, cext_hbm, t2_hbm, sr_hbm,
                     ap_hbm, seg_hbm, iis, jjs, ii, jj, xi0, xj0, xi1, xj1,
                     ov0, ov1, t2v, ebuf, rowi, rowb, sg0, sg1, so0, so1):
    c = lax.axis_index("c")
    s = lax.axis_index("s")
    wid = s * NC + c
    base = wid * _PPW

    xis = (xi0, xi1)
    xjs = (xj0, xj1)
    ovs = (ov0, ov1)
    sgs = (sg0, sg1)
    sos = (so0, so1)

    def compute(xi, xj, ov):
        def body(k, carry):
            for q in range(HP // 16):
                sl = pl.ds(q * 16, 16)
                sl2 = pl.ds(HP + q * 16, 16)
                s1 = xi[k, sl] + xj[k, sl2]
                s2 = xj[k, sl] + xi[k, sl2]
                ov[k, sl] = jnp.maximum(s1, 0.0) + jnp.maximum(s2, 0.0)
            return carry
        lax.fori_loop(0, _CH, body, 0, unroll=2)

    def schunk(it, carry):
        off = base + jnp.minimum(it * _SCP, _PPW - _SCP)
        pltpu.sync_copy(ai_hbm.at[pl.ds(off, _SCP)], iis)
        pltpu.sync_copy(aj_hbm.at[pl.ds(off, _SCP)], jjs)

        def fire(q):
            b = q % 2
            sl = pl.ds(q * _CH, _CH)
            pltpu.async_copy(x_hbm.at[iis.at[sl]], xis[b], sgs[b])
            pltpu.async_copy(x_hbm.at[jjs.at[sl]], xjs[b], sgs[b])

        fire(0)
        for q in range(_SC8):
            b = q % 2
            if q + 1 < _SC8:
                fire(q + 1)
            sl = pl.ds(q * _CH, _CH)
            pltpu.make_async_copy(x_hbm.at[iis.at[sl]], xis[b], sgs[b]).wait()
            pltpu.make_async_copy(x_hbm.at[jjs.at[sl]], xjs[b], sgs[b]).wait()
            if q >= 2:
                pltpu.make_async_copy(
                    ovs[b], ap_hbm.at[pl.ds(off + (q - 2) * _CH, _CH)], sos[b]
                ).wait()
            compute(xis[b], xjs[b], ovs[b])
            pltpu.async_copy(ovs[b], ap_hbm.at[pl.ds(off + q * _CH, _CH)], sos[b])
        pltpu.make_async_copy(
            ovs[0], ap_hbm.at[pl.ds(off + (_SC8 - 2) * _CH, _CH)], sos[0]).wait()
        pltpu.make_async_copy(
            ovs[1], ap_hbm.at[pl.ds(off + (_SC8 - 1) * _CH, _CH)], sos[1]).wait()
        return carry

    lax.fori_loop(0, _NSC, schunk, 0)

    gbase = wid * _GPT
    pltpu.sync_copy(t2_hbm, t2v)
    lanes = lax.iota(jnp.int32, 16)

    def equery(v, carry):
        s_vec = gbase + v * 16 + lanes
        j = jnp.zeros((16,), jnp.int32)
        step = 1 << 14
        while step > 0:
            nj = j + step
            idx = jnp.minimum(nj - 1, _T2N - 1)
            val = plsc.load_gather(t2v, [idx])
            ok = (nj <= _T2N) & (val < s_vec)
            j = jnp.where(ok, nj, j)
            step >>= 1
        base32 = jnp.maximum(_T2S * j - _T2S, 0)
        rowi[...] = base32 >> 7
        pltpu.async_copy(sr_hbm.at[rowi], rowb, sg0).wait()
        col0 = base32 & 127
        acc = jnp.zeros((16,), jnp.int32)
        for k in range(16):
            rk = jnp.full((16,), k, jnp.int32)
            colk = _take16(col0, rk)
            sk = _take16(s_vec, rk)
            bk = _take16(base32, rk)
            w0 = plsc.load_gather(rowb, [rk, colk + lanes])
            w1 = plsc.load_gather(rowb, [rk, colk + 16 + lanes])
            cnt = (plsc.all_reduce_population_count(w0 < sk)
                   + plsc.all_reduce_population_count(w1 < sk))
            acc = jnp.where(lanes == rk, bk + cnt, acc)
        ebuf[pl.ds(v * 16, 16)] = acc
        return carry

    lax.fori_loop(0, _NQV, equery, 0)

    def gchunk(it, carry):
        loff = jnp.minimum(it * _CH, _GPT - _CH)
        for q in range(_CH // 16):
            sl = pl.ds(q * 16, 16)
            ii[sl] = ebuf[pl.ds(loff + q * 16, 16)]
            jj[sl] = ebuf[pl.ds(loff + q * 16 + 1, 16)]
        ca = pltpu.async_copy(cext_hbm.at[ii], xi0, sg0)
        cb = pltpu.async_copy(cext_hbm.at[jj], xj0, sg0)
        ca.wait()
        cb.wait()

        def body(k, carry2):
            for q in range(HP // 16):
                sl = pl.ds(q * 16, 16)
                ov0[k, sl] = xj0[k, sl] - xi0[k, sl]
            return carry2
        lax.fori_loop(0, _CH, body, 0, unroll=2)
        pltpu.sync_copy(ov0, seg_hbm.at[pl.ds(gbase + loff, _CH)])
        return carry

    lax.fori_loop(0, _GNC, gchunk, 0)


def kernel(atom_features, pair_features, pair_split, atom_to_pair,
           W_AA, b_AA, W_PA, b_PA, W_A, b_A,
           W_AP, b_AP, W_PP, b_PP, W_P, b_P):
    wcat = jnp.zeros((FA, 3 * HP), f32)
    wcat = wcat.at[:, 0:H].set(W_AA)
    wcat = wcat.at[:, HP:HP + H].set(W_AP[:FA])
    wcat = wcat.at[:, 2 * HP:2 * HP + H].set(W_AP[FA:])
    bcat = jnp.zeros((1, 3 * HP), f32)
    bcat = bcat.at[0, 0:H].set(b_AA)
    bcat = bcat.at[0, HP:HP + H].set(0.5 * b_AP)
    bcat = bcat.at[0, 2 * HP:2 * HP + H].set(0.5 * b_AP)

    w2cat = jnp.zeros((FP, 2 * HP), f32)
    w2cat = w2cat.at[:, 0:H].set(W_PA)
    w2cat = w2cat.at[:, HP:HP + H].set(W_PP)
    b2cat = jnp.zeros((1, 2 * HP), f32)
    b2cat = b2cat.at[0, 0:H].set(b_PA)
    b2cat = b2cat.at[0, HP:HP + H].set(b_PP)

    wp2 = jnp.zeros((HP, HP), f32)
    wp2 = wp2.at[:H, :H].set(W_P[H:])
    wp1 = jnp.zeros((HP, H), f32)
    wp1 = wp1.at[:H].set(W_P[:H])
    bp = b_P.reshape(1, H)

    wa1 = jnp.zeros((HP, H), f32)
    wa1 = wa1.at[:H].set(W_A[:H])
    wa2 = jnp.zeros((HP, H), f32)
    wa2 = wa2.at[:H].set(W_A[H:])
    ba = b_A.reshape(1, H)

    ai = atom_to_pair[:, 0].astype(jnp.int32)
    aj = atom_to_pair[:, 1].astype(jnp.int32)
    split = pair_split.astype(jnp.int32)

    t2 = split[::_T2S]
    sr = split.reshape(_SR, 128)

    aa, x = _atom_pre(atom_features, wcat, bcat)
    cext, pp = _pair_pre(pair_features, w2cat, b2cat, wp2)
    ap, seg = _sc_pairs_kernel(x, ai, aj, cext, t2, sr)
    P = _pair_out(ap, pp, wp1, bp)
    A = _atom_out(aa, seg, wa1, wa2, ba)
    return (A, P)
